# Initial kernel scaffold; baseline (speedup 1.0000x reference)
import jax
import jax.numpy as jnp
from jax.experimental import pallas as pl


def kernel(x, edge_index, edge_attr, W_msg, W_edge, b_msg, W_self, W_gate, W_val):
    raise NotImplementedError("write your pallas kernel here")


# speedup vs baseline: 2.9180x; 2.9180x over previous
import functools

import jax
import jax.numpy as jnp
from jax import lax
from jax.experimental import pallas as pl
from jax.experimental.pallas import tpu as pltpu
from jax.experimental.pallas import tpu_sc as plsc

NC = 2
NS = 16
LANES = 16


def _xm_body(x_ref, w_ref, o_ref):
    o_ref[...] = jnp.dot(x_ref[...], w_ref[...],
                         preferred_element_type=jnp.float32)


def _em_body(ea_ref, w_ref, b_ref, o_ref):
    o_ref[...] = jnp.dot(ea_ref[...], w_ref[...],
                         preferred_element_type=jnp.float32) + b_ref[...]


def _epilogue_body(p0_ref, p1_ref, x_ref, ws_ref, wg_ref, wv_ref, o_ref):
    h = p0_ref[0] + p1_ref[0] + jnp.dot(
        x_ref[...], ws_ref[...], preferred_element_type=jnp.float32)
    g = jnp.dot(h, wg_ref[...], preferred_element_type=jnp.float32)
    v = jnp.dot(h, wv_ref[...], preferred_element_type=jnp.float32)
    o_ref[...] = jax.nn.sigmoid(g) * jnp.tanh(v)


def _make_sc_agg(n_pad, n_edges, d_hid, chunk):
    edges_per_tile = n_edges // (NC * NS)
    n_chunks = edges_per_tile // chunk
    rows_per_tile = n_pad // NS
    zrows = 128
    n_zcopies = rows_per_tile // zrows

    mesh = plsc.VectorSubcoreMesh(
        core_axis_name="c", subcore_axis_name="s",
        num_cores=NC, num_subcores=NS)

    @functools.partial(
        pl.kernel,
        mesh=mesh,
        out_type=jax.ShapeDtypeStruct((NC, n_pad, d_hid), jnp.float32),
        scratch_types=[
            pltpu.VMEM_SHARED((n_pad, d_hid), jnp.float32),
            pltpu.VMEM((chunk,), jnp.int32),
            pltpu.VMEM((chunk,), jnp.int32),
            pltpu.VMEM((chunk, d_hid), jnp.float32),
            pltpu.VMEM((chunk, d_hid), jnp.float32),
            pltpu.VMEM((zrows, d_hid), jnp.float32),
            pltpu.SemaphoreType.DMA,
        ],
    )
    def sc_agg(xm_hbm, em_hbm, src_hbm, dst_hbm, out_hbm,
               agg_sh, src_v, dst_v, rows_v, em_v, zbuf, sem):
        c = lax.axis_index("c")
        s = lax.axis_index("s")

        def zero_row(r, _):
            for j in range(d_hid // LANES):
                zbuf[r, pl.ds(j * LANES, LANES)] = jnp.zeros(
                    (LANES,), jnp.float32)
            return _
        lax.fori_loop(0, zrows, zero_row, 0)
        row0 = s * rows_per_tile
        for z in range(n_zcopies):
            pltpu.sync_copy(zbuf, agg_sh.at[pl.ds(row0 + z * zrows, zrows)])
        plsc.subcore_barrier()

        base_e = (c * NS + s) * edges_per_tile

        def chunk_body(k, _):
            eoff = pl.multiple_of(base_e + k * chunk, 8)
            pltpu.sync_copy(src_hbm.at[pl.ds(eoff, chunk)], src_v)
            pltpu.sync_copy(dst_hbm.at[pl.ds(eoff, chunk)], dst_v)
            gat = pltpu.async_copy(xm_hbm.at[src_v], rows_v, sem)
            pltpu.sync_copy(em_hbm.at[pl.ds(eoff, chunk)], em_v)
            gat.wait()

            def relu_row(r, carry):
                for j in range(d_hid // LANES):
                    sl = pl.ds(j * LANES, LANES)
                    rows_v[r, sl] = jnp.maximum(
                        rows_v[r, sl] + em_v[r, sl], 0.0)
                return carry
            lax.fori_loop(0, chunk, relu_row, 0)

            pltpu.sync_copy(rows_v, agg_sh.at[dst_v], add=True)
            return _
        lax.fori_loop(0, n_chunks, chunk_body, 0)

        plsc.subcore_barrier()
        pltpu.sync_copy(agg_sh.at[pl.ds(row0, rows_per_tile)],
                        out_hbm.at[c, pl.ds(row0, rows_per_tile)])

    return sc_agg


def kernel(x, edge_index, edge_attr, W_msg, W_edge, b_msg, W_self, W_gate,
           W_val):
    n_nodes, d_feat = x.shape
    n_edges = edge_index.shape[1]
    d_edge = edge_attr.shape[1]
    d_hid = W_msg.shape[1]
    d_out = W_gate.shape[1]

    src = edge_index[0]
    dst = edge_index[1]

    bn = 2000
    xm = pl.pallas_call(
        _xm_body,
        grid=(n_nodes // bn,),
        in_specs=[pl.BlockSpec((bn, d_feat), lambda i: (i, 0)),
                  pl.BlockSpec((d_feat, d_hid), lambda i: (0, 0))],
        out_specs=pl.BlockSpec((bn, d_hid), lambda i: (i, 0)),
        out_shape=jax.ShapeDtypeStruct((n_nodes, d_hid), jnp.float32),
    )(x, W_msg)

    be = 3200
    em = pl.pallas_call(
        _em_body,
        grid=(n_edges // be,),
        in_specs=[pl.BlockSpec((be, d_edge), lambda i: (i, 0)),
                  pl.BlockSpec((d_edge, d_hid), lambda i: (0, 0)),
                  pl.BlockSpec((1, d_hid), lambda i: (0, 0))],
        out_specs=pl.BlockSpec((be, d_hid), lambda i: (i, 0)),
        out_shape=jax.ShapeDtypeStruct((n_edges, d_hid), jnp.float32),
    )(edge_attr, W_edge, b_msg.reshape(1, d_hid))

    n_pad = -(-n_nodes // (NS * 128)) * (NS * 128)
    sc_agg = _make_sc_agg(n_pad, n_edges, d_hid, chunk=80)
    partials = sc_agg(xm, em, src, dst)

    out = pl.pallas_call(
        _epilogue_body,
        grid=(n_nodes // bn,),
        in_specs=[pl.BlockSpec((1, bn, d_hid), lambda i: (0, i, 0)),
                  pl.BlockSpec((1, bn, d_hid), lambda i: (1, i, 0)),
                  pl.BlockSpec((bn, d_feat), lambda i: (i, 0)),
                  pl.BlockSpec((d_feat, d_hid), lambda i: (0, 0)),
                  pl.BlockSpec((d_hid, d_out), lambda i: (0, 0)),
                  pl.BlockSpec((d_hid, d_out), lambda i: (0, 0))],
        out_specs=pl.BlockSpec((bn, d_out), lambda i: (i, 0)),
        out_shape=jax.ShapeDtypeStruct((n_nodes, d_out), jnp.float32),
    )(partials, partials, x, W_self, W_gate, W_val)
    return out


# speedup vs baseline: 3.7224x; 1.2756x over previous
import functools

import jax
import jax.numpy as jnp
from jax import lax
from jax.experimental import pallas as pl
from jax.experimental.pallas import tpu as pltpu
from jax.experimental.pallas import tpu_sc as plsc

NC = 2
NS = 16
LANES = 16


def _xm_body(x_ref, w_ref, o_ref):
    o_ref[...] = jnp.dot(x_ref[...], w_ref[...],
                         preferred_element_type=jnp.float32)


def _em_body(ea_ref, w_ref, b_ref, o_ref):
    o_ref[...] = jnp.dot(ea_ref[...], w_ref[...],
                         preferred_element_type=jnp.float32) + b_ref[...]


def _epilogue_body(p0_ref, p1_ref, x_ref, ws_ref, wg_ref, wv_ref, o_ref):
    h = p0_ref[0] + p1_ref[0] + jnp.dot(
        x_ref[...], ws_ref[...], preferred_element_type=jnp.float32)
    g = jnp.dot(h, wg_ref[...], preferred_element_type=jnp.float32)
    v = jnp.dot(h, wv_ref[...], preferred_element_type=jnp.float32)
    o_ref[...] = jax.nn.sigmoid(g) * jnp.tanh(v)


def _make_sc_agg(n_pad, n_edges, d_hid, chunk):
    edges_per_tile = n_edges // (NC * NS)
    n_chunks = edges_per_tile // chunk
    rows_per_tile = n_pad // NS
    n_zcopies = rows_per_tile // chunk

    mesh = plsc.VectorSubcoreMesh(
        core_axis_name="c", subcore_axis_name="s",
        num_cores=NC, num_subcores=NS)

    @functools.partial(
        pl.kernel,
        mesh=mesh,
        out_type=jax.ShapeDtypeStruct((NC, n_pad, d_hid), jnp.float32),
        scratch_types=[
            pltpu.VMEM_SHARED((n_pad, d_hid), jnp.float32),
            pltpu.VMEM((edges_per_tile,), jnp.int32),
            pltpu.VMEM((2, chunk), jnp.int32),
            pltpu.VMEM((2, chunk, d_hid), jnp.float32),
            pltpu.VMEM((2, chunk, d_hid), jnp.float32),
            pltpu.SemaphoreType.DMA,
            pltpu.SemaphoreType.DMA,
        ],
    )
    def sc_agg(xm_hbm, em_hbm, src_hbm, dst_hbm, out_hbm,
               agg_sh, src_slab, dst_idx, rows_v, em_v,
               gsem0, gsem1):
        c = lax.axis_index("c")
        s = lax.axis_index("s")
        wid = c * NS + s

        def zero_row(r, _):
            for j in range(d_hid // LANES):
                em_v[0, r, pl.ds(j * LANES, LANES)] = jnp.zeros(
                    (LANES,), jnp.float32)
            return _
        lax.fori_loop(0, chunk, zero_row, 0)
        row0 = s * rows_per_tile
        for z in range(n_zcopies):
            pltpu.sync_copy(em_v.at[0],
                            agg_sh.at[pl.ds(row0 + z * chunk, chunk)])
        plsc.subcore_barrier()

        base_e = wid * edges_per_tile
        pltpu.sync_copy(src_hbm.at[pl.ds(base_e, edges_per_tile)], src_slab)

        sems = (gsem0, gsem1)

        def start_fetch(k, b):
            pltpu.async_copy(xm_hbm.at[src_slab.at[pl.ds(k * chunk, chunk)]],
                             rows_v.at[b], sems[b])
            eoff = pl.multiple_of(base_e + k * chunk, 8)
            pltpu.async_copy(em_hbm.at[pl.ds(eoff, chunk)], em_v.at[b],
                             sems[b])
            pltpu.async_copy(dst_hbm.at[pl.ds(eoff, chunk)], dst_idx.at[b],
                             sems[b])

        def wait_fetch(b):
            pltpu.make_async_copy(
                xm_hbm.at[src_slab.at[pl.ds(0, chunk)]],
                rows_v.at[b], sems[b]).wait()
            pltpu.make_async_copy(em_hbm.at[pl.ds(base_e, chunk)],
                                  em_v.at[b], sems[b]).wait()
            pltpu.make_async_copy(dst_hbm.at[pl.ds(base_e, chunk)],
                                  dst_idx.at[b], sems[b]).wait()

        def half_step(k, b):
            wait_fetch(b)

            @pl.when(k + 1 < n_chunks)
            def _():
                start_fetch(k + 1, 1 - b)

            def relu_row(r, carry):
                for j in range(d_hid // LANES):
                    sl = pl.ds(j * LANES, LANES)
                    rows_v[b, r, sl] = jnp.maximum(
                        rows_v[b, r, sl] + em_v[b, r, sl], 0.0)
                return carry
            lax.fori_loop(0, chunk, relu_row, 0)

            pltpu.sync_copy(rows_v.at[b], agg_sh.at[dst_idx.at[b]],
                            add=True)

        start_fetch(0, 0)

        def pair_body(t, carry_in):
            half_step(2 * t, 0)
            half_step(2 * t + 1, 1)
            return carry_in
        lax.fori_loop(0, n_chunks // 2, pair_body, 0)

        plsc.subcore_barrier()
        pltpu.sync_copy(agg_sh.at[pl.ds(row0, rows_per_tile)],
                        out_hbm.at[c, pl.ds(row0, rows_per_tile)])

    return sc_agg


def kernel(x, edge_index, edge_attr, W_msg, W_edge, b_msg, W_self, W_gate,
           W_val):
    n_nodes, d_feat = x.shape
    n_edges = edge_index.shape[1]
    d_edge = edge_attr.shape[1]
    d_hid = W_msg.shape[1]
    d_out = W_gate.shape[1]

    src = edge_index[0]
    dst = edge_index[1]

    bn = 2000
    xm = pl.pallas_call(
        _xm_body,
        grid=(n_nodes // bn,),
        in_specs=[pl.BlockSpec((bn, d_feat), lambda i: (i, 0)),
                  pl.BlockSpec((d_feat, d_hid), lambda i: (0, 0))],
        out_specs=pl.BlockSpec((bn, d_hid), lambda i: (i, 0)),
        out_shape=jax.ShapeDtypeStruct((n_nodes, d_hid), jnp.float32),
    )(x, W_msg)

    be = 3200
    em = pl.pallas_call(
        _em_body,
        grid=(n_edges // be,),
        in_specs=[pl.BlockSpec((be, d_edge), lambda i: (i, 0)),
                  pl.BlockSpec((d_edge, d_hid), lambda i: (0, 0)),
                  pl.BlockSpec((1, d_hid), lambda i: (0, 0))],
        out_specs=pl.BlockSpec((be, d_hid), lambda i: (i, 0)),
        out_shape=jax.ShapeDtypeStruct((n_edges, d_hid), jnp.float32),
    )(edge_attr, W_edge, b_msg.reshape(1, d_hid))

    n_pad = -(-n_nodes // (NS * 128)) * (NS * 128)
    sc_agg = _make_sc_agg(n_pad, n_edges, d_hid, chunk=40)
    partials = sc_agg(xm, em, src, dst)

    out = pl.pallas_call(
        _epilogue_body,
        grid=(n_nodes // bn,),
        in_specs=[pl.BlockSpec((1, bn, d_hid), lambda i: (0, i, 0)),
                  pl.BlockSpec((1, bn, d_hid), lambda i: (1, i, 0)),
                  pl.BlockSpec((bn, d_feat), lambda i: (i, 0)),
                  pl.BlockSpec((d_feat, d_hid), lambda i: (0, 0)),
                  pl.BlockSpec((d_hid, d_out), lambda i: (0, 0)),
                  pl.BlockSpec((d_hid, d_out), lambda i: (0, 0))],
        out_specs=pl.BlockSpec((bn, d_out), lambda i: (i, 0)),
        out_shape=jax.ShapeDtypeStruct((n_nodes, d_out), jnp.float32),
    )(partials, partials, x, W_self, W_gate, W_val)
    return out
